# baseline (speedup 1.0000x reference)
import functools

import jax
import jax.numpy as jnp
from jax import lax
from jax.experimental import pallas as pl
from jax.experimental.pallas import tpu as pltpu
from jax.experimental.pallas import tpu_sc as plsc

_NC, _NS, _L = 2, 16, 16
_NW = _NC * _NS
_NPAD = 10240
_PPT = _NPAD // _NS
_CW = 128
_CH = 80
_EPW = _CH * _CW
_EPAD = _NW * _EPW


def _sc_segment_sum(row_p, dist_p):
    mesh = plsc.VectorSubcoreMesh(core_axis_name="c", subcore_axis_name="s")

    @functools.partial(
        pl.kernel,
        out_type=jax.ShapeDtypeStruct((_NC, _NPAD), jnp.float32),
        mesh=mesh,
        scratch_types=[
            pltpu.VMEM((_CH, _CW), jnp.int32),
            pltpu.VMEM((_CH, _CW), jnp.float32),
            pltpu.VMEM((_PPT,), jnp.float32),
            pltpu.VMEM_SHARED((_NPAD,), jnp.float32),
            pltpu.SemaphoreType.DMA,
        ],
    )
    def seg_sum(row_hbm, dist_hbm, out_hbm, idx_v, val_v, zero_v, agg_sh, sem):
        c = lax.axis_index("c")
        s = lax.axis_index("s")
        wid = c * _NS + s
        for i in range(_PPT // _L):
            zero_v[pl.ds(i * _L, _L)] = jnp.zeros((_L,), jnp.float32)
        pltpu.sync_copy(zero_v, agg_sh.at[pl.ds(s * _PPT, _PPT)])
        pltpu.sync_copy(row_hbm.at[wid], idx_v)
        pltpu.sync_copy(dist_hbm.at[wid], val_v)
        plsc.subcore_barrier()
        copies = [
            pltpu.async_copy(val_v.at[j], agg_sh.at[idx_v.at[j]], sem, add=True)
            for j in range(_CH)
        ]
        for cp in copies:
            cp.wait()
        plsc.subcore_barrier()
        pltpu.sync_copy(agg_sh.at[pl.ds(s * _PPT, _PPT)],
                        out_hbm.at[c, pl.ds(s * _PPT, _PPT)])

    return seg_sum(row_p, dist_p)


_BR = 1000


def _tc_mm1(h, w1a, b1):
    def body(h_ref, w1a_ref, b1_ref, o_ref):
        o_ref[...] = (jnp.dot(h_ref[...], w1a_ref[...],
                              preferred_element_type=jnp.float32) + b1_ref[...])

    n = h.shape[0]
    return pl.pallas_call(
        body,
        grid=(n // _BR,),
        in_specs=[
            pl.BlockSpec((_BR, 128), lambda i: (i, 0)),
            pl.BlockSpec((128, 128), lambda i: (0, 0)),
            pl.BlockSpec((1, 128), lambda i: (0, 0)),
        ],
        out_specs=pl.BlockSpec((_BR, 128), lambda i: (i, 0)),
        out_shape=jax.ShapeDtypeStruct((n, 128), jnp.float32),
    )(h, w1a, b1)


def _tc_mm2(h, p, aggcol, w1b, w2, b2):
    def body(h_ref, p_ref, a_ref, w1b_ref, w2_ref, b2_ref, o_ref):
        pre = p_ref[...] + a_ref[...] * w1b_ref[...]
        t = pre * jax.nn.sigmoid(pre)
        o_ref[...] = (h_ref[...]
                      + jnp.dot(t, w2_ref[...], preferred_element_type=jnp.float32)
                      + b2_ref[...])

    n = h.shape[0]
    return pl.pallas_call(
        body,
        grid=(n // _BR,),
        in_specs=[
            pl.BlockSpec((_BR, 128), lambda i: (i, 0)),
            pl.BlockSpec((_BR, 128), lambda i: (i, 0)),
            pl.BlockSpec((_BR, 1), lambda i: (i, 0)),
            pl.BlockSpec((1, 128), lambda i: (0, 0)),
            pl.BlockSpec((128, 128), lambda i: (0, 0)),
            pl.BlockSpec((1, 128), lambda i: (0, 0)),
        ],
        out_specs=pl.BlockSpec((_BR, 128), lambda i: (i, 0)),
        out_shape=jax.ShapeDtypeStruct((n, 128), jnp.float32),
    )(h, p, aggcol, w1b, w2, b2)


def kernel(h, edges, distances, W_edg1, b_edg1, W_edg2, b_edg2, W_edgi, b_edgi,
           W_node1, b_node1, W_node2, b_node2):
    n = h.shape[0]
    row = edges[0].astype(jnp.int32)
    dist = distances.reshape(-1).astype(jnp.float32)
    pad = _EPAD - row.shape[0]
    row_p = jnp.concatenate(
        [row, jnp.full((pad,), _NPAD - 1, jnp.int32)]).reshape(_NW, _CH, _CW)
    dist_p = jnp.concatenate(
        [dist, jnp.zeros((pad,), jnp.float32)]).reshape(_NW, _CH, _CW)

    parts = _sc_segment_sum(row_p, dist_p)
    aggcol = (parts[0, :n] + parts[1, :n]).reshape(n, 1)

    w1a = W_node1[:128]
    w1b = (W_node1[128] / 100.0).reshape(1, 128)
    b1 = b_node1.reshape(1, 128)
    b2 = b_node2.reshape(1, 128)

    p1 = _tc_mm1(h, w1a, b1)
    return _tc_mm2(h, p1, aggcol, w1b, W_node2, b2)


# speedup vs baseline: 1.5956x; 1.5956x over previous
import functools

import jax
import jax.numpy as jnp
from jax import lax
from jax.experimental import pallas as pl
from jax.experimental.pallas import tpu as pltpu
from jax.experimental.pallas import tpu_sc as plsc

_NC, _NS, _L = 2, 16, 16
_NW = _NC * _NS
_NPAD = 10240
_PPT = _NPAD // _NS
_CW = 128
_NCHUNK = 2560
_CPW = _NCHUNK // _NW


def _sc_segment_sum(row3, dist3):
    mesh = plsc.VectorSubcoreMesh(core_axis_name="c", subcore_axis_name="s")

    @functools.partial(
        pl.kernel,
        out_type=jax.ShapeDtypeStruct((_NC, _NPAD), jnp.float32),
        mesh=mesh,
        scratch_types=[
            pltpu.VMEM((_CPW, _CW), jnp.int32),
            pltpu.VMEM((_CPW, _CW), jnp.float32),
            pltpu.VMEM((_PPT,), jnp.float32),
            pltpu.VMEM_SHARED((_NPAD,), jnp.float32),
            pltpu.SemaphoreType.DMA,
        ],
    )
    def seg_sum(row_hbm, dist_hbm, out_hbm, idx_v, val_v, zero_v, agg_sh, sem):
        c = lax.axis_index("c")
        s = lax.axis_index("s")
        wid = c * _NS + s
        c0 = pl.multiple_of(wid * _CPW, 8)
        for i in range(_PPT // _L):
            zero_v[pl.ds(i * _L, _L)] = jnp.zeros((_L,), jnp.float32)
        pltpu.sync_copy(zero_v, agg_sh.at[pl.ds(s * _PPT, _PPT)])
        pltpu.sync_copy(row_hbm.at[pl.ds(c0, _CPW), 0], idx_v)
        pltpu.sync_copy(dist_hbm.at[pl.ds(c0, _CPW)], val_v)
        plsc.subcore_barrier()
        copies = [
            pltpu.async_copy(val_v.at[j], agg_sh.at[idx_v.at[j]], sem, add=True)
            for j in range(_CPW)
        ]
        for cp in copies:
            cp.wait()
        plsc.subcore_barrier()
        pltpu.sync_copy(agg_sh.at[pl.ds(s * _PPT, _PPT)],
                        out_hbm.at[c, pl.ds(s * _PPT, _PPT)])

    return seg_sum(row3, dist3)


_BR = 2048


def _tc_mm1(h, w1a, b1):
    def body(h_ref, w1a_ref, b1_ref, o_ref):
        o_ref[...] = (jnp.dot(h_ref[...], w1a_ref[...],
                              preferred_element_type=jnp.float32) + b1_ref[...])

    n = h.shape[0]
    return pl.pallas_call(
        body,
        grid=(pl.cdiv(n, _BR),),
        in_specs=[
            pl.BlockSpec((_BR, 128), lambda i: (i, 0)),
            pl.BlockSpec((128, 128), lambda i: (0, 0)),
            pl.BlockSpec((1, 128), lambda i: (0, 0)),
        ],
        out_specs=pl.BlockSpec((_BR, 128), lambda i: (i, 0)),
        out_shape=jax.ShapeDtypeStruct((n, 128), jnp.float32),
    )(h, w1a, b1)


def _tc_mm2(h, p, parts, w1b2, w2, b2):
    def body(h_ref, p_ref, pt_ref, w1b2_ref, w2_ref, b2_ref, o_ref):
        pre = p_ref[...] + lax.dot_general(
            pt_ref[...], w1b2_ref[...], (((0,), (0,)), ((), ())),
            preferred_element_type=jnp.float32)
        t = pre * jax.nn.sigmoid(pre)
        o_ref[...] = (h_ref[...]
                      + jnp.dot(t, w2_ref[...], preferred_element_type=jnp.float32)
                      + b2_ref[...])

    n = h.shape[0]
    return pl.pallas_call(
        body,
        grid=(pl.cdiv(n, _BR),),
        in_specs=[
            pl.BlockSpec((_BR, 128), lambda i: (i, 0)),
            pl.BlockSpec((_BR, 128), lambda i: (i, 0)),
            pl.BlockSpec((_NC, _BR), lambda i: (0, i)),
            pl.BlockSpec((_NC, 128), lambda i: (0, 0)),
            pl.BlockSpec((128, 128), lambda i: (0, 0)),
            pl.BlockSpec((1, 128), lambda i: (0, 0)),
        ],
        out_specs=pl.BlockSpec((_BR, 128), lambda i: (i, 0)),
        out_shape=jax.ShapeDtypeStruct((n, 128), jnp.float32),
    )(h, p, parts, w1b2, w2, b2)


def kernel(h, edges, distances, W_edg1, b_edg1, W_edg2, b_edg2, W_edgi, b_edgi,
           W_node1, b_node1, W_node2, b_node2):
    pad = _NCHUNK * _CW - edges.shape[1]
    row3 = (jnp.pad(edges.astype(jnp.int32), ((0, 0), (0, pad)))
            .reshape(2, _NCHUNK, _CW).transpose(1, 0, 2))
    dist3 = jnp.pad(distances, ((0, pad), (0, 0))).reshape(_NCHUNK, _CW)

    parts = _sc_segment_sum(row3, dist3)

    w1a = W_node1[:128]
    w1b2 = jnp.tile((W_node1[128] / 100.0).reshape(1, 128), (_NC, 1))
    b1 = b_node1.reshape(1, 128)
    b2 = b_node2.reshape(1, 128)

    p1 = _tc_mm1(h, w1a, b1)
    return _tc_mm2(h, p1, parts, w1b2, W_node2, b2)


# speedup vs baseline: 1.7809x; 1.1162x over previous
import functools

import jax
import jax.numpy as jnp
from jax import lax
from jax.experimental import pallas as pl
from jax.experimental.pallas import tpu as pltpu
from jax.experimental.pallas import tpu_sc as plsc

_NC, _NS, _L = 2, 16, 16
_NW = _NC * _NS
_NPAD = 10240
_PPT = _NPAD // _NS
_CW = 128
_NCHUNK = 2500
_CPW = 80
_CLAST = _NCHUNK - (_NW - 1) * _CPW
_G = 4
_GPW = _CPW // _G
_GLAST = _CLAST // _G


def _sc_segment_sum(row3, dist4):
    mesh = plsc.VectorSubcoreMesh(core_axis_name="c", subcore_axis_name="s")

    @functools.partial(
        pl.kernel,
        out_type=jax.ShapeDtypeStruct((_NC, _NPAD), jnp.float32),
        mesh=mesh,
        scratch_types=[
            pltpu.VMEM((_CPW, _CW), jnp.int32),
            pltpu.VMEM((_GPW, _G, _CW), jnp.float32),
            pltpu.VMEM((_PPT,), jnp.float32),
            pltpu.VMEM_SHARED((_NPAD,), jnp.float32),
            pltpu.SemaphoreType.DMA,
        ],
    )
    def seg_sum(row_hbm, dist_hbm, out_hbm, idx_v, val_v, zero_v, agg_sh, sem):
        c = lax.axis_index("c")
        s = lax.axis_index("s")
        wid = c * _NS + s
        c0 = pl.multiple_of(wid * _CPW, 8)
        last = wid == _NW - 1
        for i in range(_PPT // _L):
            zero_v[pl.ds(i * _L, _L)] = jnp.zeros((_L,), jnp.float32)
        pltpu.sync_copy(zero_v, agg_sh.at[pl.ds(s * _PPT, _PPT)])

        g0 = wid * _GPW
        @pl.when(jnp.logical_not(last))
        def _():
            pltpu.sync_copy(row_hbm.at[pl.ds(c0, _CPW), 0], idx_v)
            pltpu.sync_copy(dist_hbm.at[pl.ds(g0, _GPW)], val_v)

        @pl.when(last)
        def _():
            pltpu.sync_copy(row_hbm.at[pl.ds((_NW - 1) * _CPW, _CLAST), 0],
                            idx_v.at[pl.ds(0, _CLAST)])
            pltpu.sync_copy(dist_hbm.at[pl.ds((_NW - 1) * _GPW, _GLAST)],
                            val_v.at[pl.ds(0, _GLAST)])
            sink = jnp.full((_L,), _NPAD - 1, jnp.int32)
            for j in range(_CLAST, _CPW):
                for i in range(_CW // _L):
                    idx_v[j, pl.ds(i * _L, _L)] = sink

        plsc.subcore_barrier()
        copies = [
            pltpu.async_copy(val_v.at[j // _G, j % _G],
                             agg_sh.at[idx_v.at[j]], sem, add=True)
            for j in range(_CPW)
        ]
        for cp in copies:
            cp.wait()
        plsc.subcore_barrier()
        pltpu.sync_copy(agg_sh.at[pl.ds(s * _PPT, _PPT)],
                        out_hbm.at[c, pl.ds(s * _PPT, _PPT)])

    return seg_sum(row3, dist4)


_BR = 2048


def _tc_node_mlp(h, parts, w1a, w1b2, b1, w2, b2):
    def body(h_ref, pt_ref, w1a_ref, w1b2_ref, b1_ref, w2_ref, b2_ref, o_ref):
        x = h_ref[...]
        pre = (jnp.dot(x, w1a_ref[...], preferred_element_type=jnp.float32)
               + lax.dot_general(pt_ref[...], w1b2_ref[...],
                                 (((0,), (0,)), ((), ())),
                                 preferred_element_type=jnp.float32)
               + b1_ref[...])
        t = pre * jax.nn.sigmoid(pre)
        o_ref[...] = (x + jnp.dot(t, w2_ref[...],
                                  preferred_element_type=jnp.float32)
                      + b2_ref[...])

    n = h.shape[0]
    return pl.pallas_call(
        body,
        grid=(pl.cdiv(n, _BR),),
        in_specs=[
            pl.BlockSpec((_BR, 128), lambda i: (i, 0)),
            pl.BlockSpec((_NC, _BR), lambda i: (0, i)),
            pl.BlockSpec((128, 128), lambda i: (0, 0)),
            pl.BlockSpec((_NC, 128), lambda i: (0, 0)),
            pl.BlockSpec((1, 128), lambda i: (0, 0)),
            pl.BlockSpec((128, 128), lambda i: (0, 0)),
            pl.BlockSpec((1, 128), lambda i: (0, 0)),
        ],
        out_specs=pl.BlockSpec((_BR, 128), lambda i: (i, 0)),
        out_shape=jax.ShapeDtypeStruct((n, 128), jnp.float32),
    )(h, parts, w1a, w1b2, b1, w2, b2)


def kernel(h, edges, distances, W_edg1, b_edg1, W_edg2, b_edg2, W_edgi, b_edgi,
           W_node1, b_node1, W_node2, b_node2):
    row3 = (edges.astype(jnp.int32).reshape(2, _NCHUNK, _CW)
            .transpose(1, 0, 2))
    dist4 = distances.reshape(_NCHUNK // _G, _G, _CW)

    parts = _sc_segment_sum(row3, dist4)

    w1a = W_node1[:128]
    w1b2 = jnp.tile((W_node1[128] / 100.0).reshape(1, 128), (_NC, 1))
    b1 = b_node1.reshape(1, 128)
    b2 = b_node2.reshape(1, 128)
    return _tc_node_mlp(h, parts, w1a, w1b2, b1, W_node2, b2)
